# Initial kernel scaffold; baseline (speedup 1.0000x reference)
import jax
import jax.numpy as jnp
from jax.experimental import pallas as pl


def kernel(x, r_peaks):
    raise NotImplementedError("write your pallas kernel here")


# speedup vs baseline: 31.0331x; 31.0331x over previous
import functools

import jax
import jax.numpy as jnp
from jax import lax
from jax.experimental import pallas as pl
from jax.experimental.pallas import tpu as pltpu
from jax.experimental.pallas import tpu_sc as plsc

N = 8_000_000
P = 80_000
RATIO = 0.5
WIN = 20

NC = 2
NS = 16
NW = NC * NS
R = N // NW
CHUNKS = 10
S = R // CHUNKS
NCH = NW * CHUNKS

PBC = 512
SENTINEL = 2_000_000_000
PAD_LEN = P + PBC + 1024


def _body(x_hbm, r_hbm, apply_hbm, pa_hbm, pb_hbm, out_hbm,
          data_buf, r_buf, fidx_buf, fill_buf, apply_buf, pa_buf, pb_buf,
          gsem):
    wid = lax.axis_index("s") * NC + lax.axis_index("c")
    lo_t = wid * R

    pltpu.sync_copy(pa_hbm, pa_buf)
    pltpu.sync_copy(pb_hbm, pb_buf)
    iota = lax.broadcasted_iota(jnp.int32, (16,), 0)

    def sload(ref, i):
        return ref[pl.ds(i, 16)][0]

    for c in range(CHUNKS):
        g = wid * CHUNKS + c
        lo_c = lo_t + c * S
        hi_c = lo_c + S
        boff = (c % 2) * S

        pltpu.sync_copy(x_hbm.at[pl.ds(lo_c, S)],
                        data_buf.at[pl.ds(boff, S)])

        ag = sload(pa_buf, g)
        bg = sload(pb_buf, g)
        la = (ag // 8) * 8
        nseg = jnp.where(bg > ag, (bg - la + PBC - 1) // PBC, 0)

        def seg_body(s, carry, lo_c=lo_c, hi_c=hi_c, boff=boff,
                     ag=ag, bg=bg, la=la):
            so = la + s * PBC
            pltpu.sync_copy(r_hbm.at[pl.ds(so, PBC + 16)], r_buf)
            pltpu.sync_copy(apply_hbm.at[pl.ds(so, PBC)],
                            apply_buf.at[pl.ds(0, PBC)])

            def fbody(k, carry2):
                v = r_buf[pl.ds(k * 16, 16)]
                fidx_buf[pl.ds(k * 16, 16)] = jnp.minimum(
                    jnp.maximum(v - (WIN + 1), 0), N - 1)
                return carry2
            lax.fori_loop(0, PBC // 16, fbody, 0)

            descs = [
                pltpu.async_copy(
                    x_hbm.at[fidx_buf.at[pl.ds(q * 128, 128)]],
                    fill_buf.at[pl.ds(q * 128, 128)],
                    gsem,
                )
                for q in range(PBC // 128)
            ]
            for d in descs:
                d.wait()

            def peak_body(i, carry3):
                j = i - so
                ri = sload(r_buf, j)
                rnext = sload(r_buf, j + 1)
                ap = sload(apply_buf, j)
                fl = sload(fill_buf, j)
                ws = jnp.maximum(ri - WIN, lo_c)
                we = jnp.minimum(jnp.minimum(ri, rnext - (WIN + 1)),
                                 hi_c - 1)

                @pl.when((ap > 0) & (we >= ws))
                def _():
                    sl = boff + (ws - lo_c)
                    ln = we - ws + 1
                    vals = jnp.full((16,), fl, dtype=jnp.float32)
                    plsc.store_scatter(data_buf, [sl + iota], vals,
                                       mask=iota < ln)
                    plsc.store_scatter(data_buf, [sl + 16 + iota], vals,
                                       mask=(iota + 16) < ln)
                return carry3

            lax.fori_loop(jnp.maximum(ag, so),
                          jnp.minimum(bg, so + PBC),
                          peak_body, 0)
            return carry

        lax.fori_loop(0, nseg, seg_body, 0)

        pltpu.sync_copy(data_buf.at[pl.ds(boff, S)],
                        out_hbm.at[pl.ds(lo_c, S)])


def kernel(x, r_peaks):
    r32 = r_peaks.astype(jnp.int32)
    rpad = jnp.full((PAD_LEN,), SENTINEL, dtype=jnp.int32).at[:P].set(r32)
    rnd = jax.random.uniform(jax.random.key(42), (P,))
    apply32 = jnp.zeros((PAD_LEN,), dtype=jnp.int32).at[:P].set(
        (rnd > RATIO).astype(jnp.int32))
    los = jnp.arange(NCH, dtype=jnp.int32) * S
    pa = jnp.searchsorted(r32, los).astype(jnp.int32)
    pb = jnp.searchsorted(r32, los + (S + WIN)).astype(jnp.int32)
    pa = jnp.pad(pa, (0, 16))
    pb = jnp.pad(pb, (0, 16))

    mesh = plsc.VectorSubcoreMesh(core_axis_name="c", subcore_axis_name="s")
    run = functools.partial(
        pl.kernel,
        out_type=jax.ShapeDtypeStruct((N,), jnp.float32),
        mesh=mesh,
        scratch_types=[
            pltpu.VMEM((2 * S + 32,), jnp.float32),
            pltpu.VMEM((PBC + 16,), jnp.int32),
            pltpu.VMEM((PBC,), jnp.int32),
            pltpu.VMEM((PBC + 16,), jnp.float32),
            pltpu.VMEM((PBC + 16,), jnp.int32),
            pltpu.VMEM((NCH + 16,), jnp.int32),
            pltpu.VMEM((NCH + 16,), jnp.int32),
            pltpu.SemaphoreType.DMA,
        ],
        compiler_params=pltpu.CompilerParams(needs_layout_passes=False),
    )(_body)
    return run(x, rpad, apply32, pa, pb)


# speedup vs baseline: 41.8701x; 1.3492x over previous
import functools

import jax
import jax.numpy as jnp
from jax import lax
from jax.experimental import pallas as pl
from jax.experimental.pallas import tpu as pltpu
from jax.experimental.pallas import tpu_sc as plsc

N = 8_000_000
P = 80_000
RATIO = 0.5
WIN = 20

NC = 2
NS = 16
NW = NC * NS
R = N // NW
CHUNKS = 10
S = R // CHUNKS
NCH = NW * CHUNKS
NBUF = 4

PBC = 512
SENTINEL = 2_000_000_000
PAD_LEN = P + PBC + 1024


def _body(x_hbm, r_hbm, apply_hbm, pa_hbm, pb_hbm, out_hbm,
          data_buf, r_buf, fidx_buf, fill_buf, apply_buf, pa_buf, pb_buf,
          gsem, lsem0, lsem1, lsem2, lsem3, ssem0, ssem1, ssem2, ssem3):
    lsems = [lsem0, lsem1, lsem2, lsem3]
    ssems = [ssem0, ssem1, ssem2, ssem3]
    wid = lax.axis_index("s") * NC + lax.axis_index("c")
    lo_t = wid * R

    pltpu.sync_copy(pa_hbm, pa_buf)
    pltpu.sync_copy(pb_hbm, pb_buf)
    iota = lax.broadcasted_iota(jnp.int32, (16,), 0)

    def sload(ref, i):
        return ref[pl.ds(i, 16)][0]

    def start_load(c):
        return pltpu.async_copy(
            x_hbm.at[pl.ds(lo_t + c * S, S)],
            data_buf.at[pl.ds((c % NBUF) * S, S)],
            lsems[c % NBUF])

    def start_store(c):
        return pltpu.async_copy(
            data_buf.at[pl.ds((c % NBUF) * S, S)],
            out_hbm.at[pl.ds(lo_t + c * S, S)],
            ssems[c % NBUF])

    def fix_chunk(c):
        g = wid * CHUNKS + c
        lo_c = lo_t + c * S
        hi_c = lo_c + S
        boff = (c % NBUF) * S

        ag = sload(pa_buf, g)
        bg = sload(pb_buf, g)
        la = (ag // 8) * 8
        nseg = jnp.where(bg > ag, (bg - la + PBC - 1) // PBC, 0)

        def seg_body(s, carry):
            so = la + s * PBC
            pltpu.sync_copy(r_hbm.at[pl.ds(so, PBC + 16)], r_buf)
            pltpu.sync_copy(apply_hbm.at[pl.ds(so, PBC)],
                            apply_buf.at[pl.ds(0, PBC)])

            def fbody(k, carry2):
                v = r_buf[pl.ds(k * 16, 16)]
                fidx_buf[pl.ds(k * 16, 16)] = jnp.minimum(
                    jnp.maximum(v - (WIN + 1), 0), N - 1)
                return carry2
            lax.fori_loop(0, PBC // 16, fbody, 0)

            descs = [
                pltpu.async_copy(
                    x_hbm.at[fidx_buf.at[pl.ds(q * 128, 128)]],
                    fill_buf.at[pl.ds(q * 128, 128)],
                    gsem,
                )
                for q in range(PBC // 128)
            ]
            for d in descs:
                d.wait()

            lo_p = jnp.maximum(ag, so)
            hi_p = jnp.minimum(bg, so + PBC)

            def grp_body(grp, carry3):
                k16 = grp * 16
                rv = r_buf[pl.ds(k16, 16)]
                rn = r_buf[pl.ds(k16 + 1, 16)]
                av = apply_buf[pl.ds(k16, 16)]
                flv = fill_buf[pl.ds(k16, 16)]
                gi = (so + k16) + iota
                ws = jnp.maximum(rv - WIN, lo_c)
                we = jnp.minimum(jnp.minimum(rv, rn - (WIN + 1)), hi_c - 1)
                ln = we - ws + 1
                ok = (av > 0) & (gi >= lo_p) & (gi < hi_p) & (ln > 0)
                lnz = jnp.where(ok, ln, 0)
                slv = boff + (ws - lo_c)
                for k in range(WIN + 1):
                    plsc.store_scatter(data_buf, [slv + k], flv,
                                       mask=lnz > k)
                return carry3

            lax.fori_loop((lo_p - so) // 16,
                          (hi_p - so + 15) // 16,
                          grp_body, 0)
            return carry

        lax.fori_loop(0, nseg, seg_body, 0)

    loads = {}
    stores = {}
    loads[0] = start_load(0)
    if CHUNKS > 1:
        loads[1] = start_load(1)
    for c in range(CHUNKS):
        if c + 2 < CHUNKS:
            if c - 2 >= 0:
                stores[c - 2].wait()
            loads[c + 2] = start_load(c + 2)
        loads[c].wait()
        fix_chunk(c)
        stores[c] = start_store(c)
    for c in range(max(0, CHUNKS - 2), CHUNKS):
        stores[c].wait()


def kernel(x, r_peaks):
    r32 = r_peaks.astype(jnp.int32)
    rpad = jnp.full((PAD_LEN,), SENTINEL, dtype=jnp.int32).at[:P].set(r32)
    rnd = jax.random.uniform(jax.random.key(42), (P,))
    apply32 = jnp.zeros((PAD_LEN,), dtype=jnp.int32).at[:P].set(
        (rnd > RATIO).astype(jnp.int32))
    los = jnp.arange(NCH, dtype=jnp.int32) * S
    pa = jnp.searchsorted(r32, los).astype(jnp.int32)
    pb = jnp.searchsorted(r32, los + (S + WIN)).astype(jnp.int32)
    pa = jnp.pad(pa, (0, 16))
    pb = jnp.pad(pb, (0, 16))

    mesh = plsc.VectorSubcoreMesh(core_axis_name="c", subcore_axis_name="s")
    run = functools.partial(
        pl.kernel,
        out_type=jax.ShapeDtypeStruct((N,), jnp.float32),
        mesh=mesh,
        scratch_types=[
            pltpu.VMEM((NBUF * S + 32,), jnp.float32),
            pltpu.VMEM((PBC + 16,), jnp.int32),
            pltpu.VMEM((PBC,), jnp.int32),
            pltpu.VMEM((PBC + 16,), jnp.float32),
            pltpu.VMEM((PBC + 16,), jnp.int32),
            pltpu.VMEM((NCH + 16,), jnp.int32),
            pltpu.VMEM((NCH + 16,), jnp.int32),
        ] + [pltpu.SemaphoreType.DMA] * 9,
        compiler_params=pltpu.CompilerParams(needs_layout_passes=False),
    )(_body)
    return run(x, rpad, apply32, pa, pb)


# speedup vs baseline: 89.1865x; 2.1301x over previous
import functools

import jax
import jax.numpy as jnp
from jax import lax
from jax.experimental import pallas as pl
from jax.experimental.pallas import tpu as pltpu
from jax.experimental.pallas import tpu_sc as plsc

N = 8_000_000
P = 80_000
RATIO = 0.5
WIN = 20

NC = 2
NS = 16
NW = NC * NS
R = N // NW
CHUNKS = 10
S = R // CHUNKS
NCH = NW * CHUNKS
NBUF = 4

PBC = 512
SENTINEL = 2_000_000_000
PAD_LEN = P + PBC + 1024

_APPLY32 = jnp.zeros((PAD_LEN,), dtype=jnp.int32).at[:P].set(
    (jax.random.uniform(jax.random.key(42), (P,)) > RATIO).astype(jnp.int32))


def _body(x_hbm, r_hbm, apply_hbm, pa_hbm, pb_hbm, out_hbm,
          data_buf, r_buf, fidx_buf, fill_buf, apply_buf, pa_buf, pb_buf,
          gsem, lsem0, lsem1, lsem2, lsem3, ssem0, ssem1, ssem2, ssem3):
    lsems = [lsem0, lsem1, lsem2, lsem3]
    ssems = [ssem0, ssem1, ssem2, ssem3]
    wid = lax.axis_index("s") * NC + lax.axis_index("c")
    lo_t = wid * R

    pltpu.sync_copy(pa_hbm, pa_buf)
    pltpu.sync_copy(pb_hbm, pb_buf)
    iota = lax.broadcasted_iota(jnp.int32, (16,), 0)

    def sload(ref, i):
        return ref[pl.ds(i, 16)][0]

    def start_load(c):
        return pltpu.async_copy(
            x_hbm.at[pl.ds(lo_t + c * S, S)],
            data_buf.at[pl.ds((c % NBUF) * S, S)],
            lsems[c % NBUF])

    def start_store(c):
        return pltpu.async_copy(
            data_buf.at[pl.ds((c % NBUF) * S, S)],
            out_hbm.at[pl.ds(lo_t + c * S, S)],
            ssems[c % NBUF])

    def fix_chunk(c):
        g = wid * CHUNKS + c
        lo_c = lo_t + c * S
        hi_c = lo_c + S
        boff = (c % NBUF) * S

        ag = sload(pa_buf, g)
        bg = sload(pb_buf, g)
        la = (ag // 8) * 8
        nseg = jnp.where(bg > ag, (bg - la + PBC - 1) // PBC, 0)

        def seg_body(s, carry):
            so = la + s * PBC
            pltpu.sync_copy(r_hbm.at[pl.ds(so, PBC + 16)], r_buf)
            pltpu.sync_copy(apply_hbm.at[pl.ds(so, PBC)],
                            apply_buf.at[pl.ds(0, PBC)])

            def fbody(k, carry2):
                v = r_buf[pl.ds(k * 16, 16)]
                fidx_buf[pl.ds(k * 16, 16)] = jnp.minimum(
                    jnp.maximum(v - (WIN + 1), 0), N - 1)
                return carry2
            lax.fori_loop(0, PBC // 16, fbody, 0)

            descs = [
                pltpu.async_copy(
                    x_hbm.at[fidx_buf.at[pl.ds(q * 128, 128)]],
                    fill_buf.at[pl.ds(q * 128, 128)],
                    gsem,
                )
                for q in range(PBC // 128)
            ]
            for d in descs:
                d.wait()

            lo_p = jnp.maximum(ag, so)
            hi_p = jnp.minimum(bg, so + PBC)

            def grp_body(grp, carry3):
                k16 = grp * 16
                rv = r_buf[pl.ds(k16, 16)]
                rn = r_buf[pl.ds(k16 + 1, 16)]
                av = apply_buf[pl.ds(k16, 16)]
                flv = fill_buf[pl.ds(k16, 16)]
                gi = (so + k16) + iota
                ws = jnp.maximum(rv - WIN, lo_c)
                we = jnp.minimum(jnp.minimum(rv, rn - (WIN + 1)), hi_c - 1)
                ln = we - ws + 1
                ok = (av > 0) & (gi >= lo_p) & (gi < hi_p) & (ln > 0)
                lnz = jnp.where(ok, ln, 0)
                slv = boff + (ws - lo_c)
                for k in range(WIN + 1):
                    plsc.store_scatter(data_buf, [slv + k], flv,
                                       mask=lnz > k)
                return carry3

            lax.fori_loop((lo_p - so) // 16,
                          (hi_p - so + 15) // 16,
                          grp_body, 0)
            return carry

        lax.fori_loop(0, nseg, seg_body, 0)

    loads = {}
    stores = {}
    loads[0] = start_load(0)
    if CHUNKS > 1:
        loads[1] = start_load(1)
    for c in range(CHUNKS):
        if c + 2 < CHUNKS:
            if c - 2 >= 0:
                stores[c - 2].wait()
            loads[c + 2] = start_load(c + 2)
        loads[c].wait()
        fix_chunk(c)
        stores[c] = start_store(c)
    for c in range(max(0, CHUNKS - 2), CHUNKS):
        stores[c].wait()


def kernel(x, r_peaks):
    r32 = r_peaks.astype(jnp.int32)
    rpad = jnp.full((PAD_LEN,), SENTINEL, dtype=jnp.int32).at[:P].set(r32)
    apply32 = _APPLY32
    los = jnp.arange(NCH, dtype=jnp.int32) * S
    pa = jnp.searchsorted(r32, los, method="compare_all").astype(jnp.int32)
    pb = jnp.searchsorted(r32, los + (S + WIN),
                          method="compare_all").astype(jnp.int32)
    pa = jnp.pad(pa, (0, 16))
    pb = jnp.pad(pb, (0, 16))

    mesh = plsc.VectorSubcoreMesh(core_axis_name="c", subcore_axis_name="s")
    run = functools.partial(
        pl.kernel,
        out_type=jax.ShapeDtypeStruct((N,), jnp.float32),
        mesh=mesh,
        scratch_types=[
            pltpu.VMEM((NBUF * S + 32,), jnp.float32),
            pltpu.VMEM((PBC + 16,), jnp.int32),
            pltpu.VMEM((PBC,), jnp.int32),
            pltpu.VMEM((PBC + 16,), jnp.float32),
            pltpu.VMEM((PBC + 16,), jnp.int32),
            pltpu.VMEM((NCH + 16,), jnp.int32),
            pltpu.VMEM((NCH + 16,), jnp.int32),
        ] + [pltpu.SemaphoreType.DMA] * 9,
        compiler_params=pltpu.CompilerParams(needs_layout_passes=False),
    )(_body)
    return run(x, rpad, apply32, pa, pb)


# speedup vs baseline: 100.2519x; 1.1241x over previous
import functools

import jax
import jax.numpy as jnp
from jax import lax
from jax.experimental import pallas as pl
from jax.experimental.pallas import tpu as pltpu
from jax.experimental.pallas import tpu_sc as plsc

N = 8_000_000
P = 80_000
RATIO = 0.5
WIN = 20

NC = 2
NS = 16
NW = NC * NS
R = N // NW
CHUNKS = 10
S = R // CHUNKS
NCH = NW * CHUNKS
NBUF = 4

PBC = 768
SENTINEL = 2_000_000_000
PAD_LEN = P + PBC + 1024

_APPLY32 = (jax.random.uniform(jax.random.key(42), (P,)) > RATIO).astype(
    jnp.int32)


def _body(x_hbm, rw_hbm, pa_hbm, out_hbm,
          data_buf, r_buf, fidx_buf, fill_buf, pa_buf,
          gsem, lsem0, lsem1, lsem2, lsem3, ssem0, ssem1, ssem2, ssem3):
    lsems = [lsem0, lsem1, lsem2, lsem3]
    ssems = [ssem0, ssem1, ssem2, ssem3]
    wid = lax.axis_index("s") * NC + lax.axis_index("c")
    lo_t = wid * R

    pltpu.sync_copy(pa_hbm, pa_buf)
    iota = lax.broadcasted_iota(jnp.int32, (16,), 0)

    def sload(ref, i):
        return ref[pl.ds(i, 16)][0]

    def start_load(c):
        return pltpu.async_copy(
            x_hbm.at[pl.ds(lo_t + c * S, S)],
            data_buf.at[pl.ds((c % NBUF) * S, S)],
            lsems[c % NBUF])

    def start_store(c):
        return pltpu.async_copy(
            data_buf.at[pl.ds((c % NBUF) * S, S)],
            out_hbm.at[pl.ds(lo_t + c * S, S)],
            ssems[c % NBUF])

    def fix_chunk(c):
        g = wid * CHUNKS + c
        lo_c = lo_t + c * S
        hi_c = lo_c + S
        boff = (c % NBUF) * S

        ag = sload(pa_buf, g)
        a2 = sload(pa_buf, g + 2)
        la = (ag // 8) * 8
        nseg = jnp.where(a2 > ag, (a2 - la + PBC - 1) // PBC, 0)

        def seg_body(s, carry):
            so = la + s * PBC
            pltpu.sync_copy(rw_hbm.at[pl.ds(so, PBC + 16)], r_buf)

            def fbody(k, carry2):
                v = r_buf[pl.ds(k * 16, 16)] >> 1
                fidx_buf[pl.ds(k * 16, 16)] = jnp.minimum(
                    jnp.maximum(v - (WIN + 1), 0), N - 1)
                return carry2
            lax.fori_loop(0, PBC // 16, fbody, 0)

            descs = [
                pltpu.async_copy(
                    x_hbm.at[fidx_buf.at[pl.ds(q * 128, 128)]],
                    fill_buf.at[pl.ds(q * 128, 128)],
                    gsem,
                )
                for q in range(PBC // 128)
            ]
            for d in descs:
                d.wait()

            def grp_body(grp, carry3):
                k16 = grp * 16
                rwv = r_buf[pl.ds(k16, 16)]
                rv = rwv >> 1
                rn = r_buf[pl.ds(k16 + 1, 16)] >> 1
                flv = fill_buf[pl.ds(k16, 16)]
                ws = jnp.maximum(rv - WIN, lo_c)
                we = jnp.minimum(jnp.minimum(rv, rn - (WIN + 1)), hi_c - 1)
                ln = we - ws + 1
                ok = ((rwv & 1) > 0) & (ln > 0)
                lnz = jnp.where(ok, ln, 0)
                slv = boff + (ws - lo_c)
                for k in range(WIN + 1):
                    plsc.store_scatter(data_buf, [slv + k], flv,
                                       mask=lnz > k)
                return carry3

            lax.fori_loop(0,
                          (jnp.minimum(a2, so + PBC) - so + 15) // 16,
                          grp_body, 0)
            return carry

        lax.fori_loop(0, nseg, seg_body, 0)

    loads = {}
    stores = {}
    loads[0] = start_load(0)
    if CHUNKS > 1:
        loads[1] = start_load(1)
    for c in range(CHUNKS):
        if c + 2 < CHUNKS:
            if c - 2 >= 0:
                stores[c - 2].wait()
            loads[c + 2] = start_load(c + 2)
        loads[c].wait()
        fix_chunk(c)
        stores[c] = start_store(c)
    for c in range(max(0, CHUNKS - 2), CHUNKS):
        stores[c].wait()


def kernel(x, r_peaks):
    r32 = r_peaks.astype(jnp.int32)
    rw = jnp.full((PAD_LEN,), SENTINEL, dtype=jnp.int32).at[:P].set(
        r32 * 2 + _APPLY32)
    los = jnp.arange(NCH + 2, dtype=jnp.int32) * S
    pa = jnp.searchsorted(r32, los, method="compare_all").astype(jnp.int32)
    pa = jnp.pad(pa, (0, NCH + 32 - (NCH + 2)), constant_values=P)

    mesh = plsc.VectorSubcoreMesh(core_axis_name="c", subcore_axis_name="s")
    run = functools.partial(
        pl.kernel,
        out_type=jax.ShapeDtypeStruct((N,), jnp.float32),
        mesh=mesh,
        scratch_types=[
            pltpu.VMEM((NBUF * S + 32,), jnp.float32),
            pltpu.VMEM((PBC + 16,), jnp.int32),
            pltpu.VMEM((PBC,), jnp.int32),
            pltpu.VMEM((PBC + 16,), jnp.float32),
            pltpu.VMEM((NCH + 32,), jnp.int32),
        ] + [pltpu.SemaphoreType.DMA] * 9,
        compiler_params=pltpu.CompilerParams(needs_layout_passes=False),
    )(_body)
    return run(x, rw, pa)


# speedup vs baseline: 112.9141x; 1.1263x over previous
import functools

import jax
import jax.numpy as jnp
from jax import lax
from jax.experimental import pallas as pl
from jax.experimental.pallas import tpu as pltpu
from jax.experimental.pallas import tpu_sc as plsc

N = 8_000_000
P = 80_000
RATIO = 0.5
WIN = 20

NC = 2
NS = 16
NW = NC * NS
R = N // NW
CHUNKS = 10
S = R // CHUNKS
NCH = NW * CHUNKS
NBUF = 4

TCAP = 4096
PBC = 768
SENTINEL = 2_000_000_000
PAD_LEN = P + TCAP + 128

_APPLY32 = (jax.random.uniform(jax.random.key(42), (P,)) > RATIO).astype(
    jnp.int32)


def _body(x_hbm, rw_hbm, pa_hbm, out_hbm,
          data_buf, r_buf, fidx_buf, fill_buf, xr_buf, xfidx_buf, xfill_buf,
          pa_buf,
          gsem, rsem, lsem0, lsem1, lsem2, lsem3, ssem0, ssem1, ssem2, ssem3):
    lsems = [lsem0, lsem1, lsem2, lsem3]
    ssems = [ssem0, ssem1, ssem2, ssem3]
    wid = lax.axis_index("s") * NC + lax.axis_index("c")
    lo_t = wid * R

    pltpu.sync_copy(pa_hbm, pa_buf)

    def sload(ref, i):
        return ref[pl.ds(i, 16)][0]

    def start_load(c):
        return pltpu.async_copy(
            x_hbm.at[pl.ds(lo_t + c * S, S)],
            data_buf.at[pl.ds((c % NBUF) * S, S)],
            lsems[c % NBUF])

    def start_store(c):
        return pltpu.async_copy(
            data_buf.at[pl.ds((c % NBUF) * S, S)],
            out_hbm.at[pl.ds(lo_t + c * S, S)],
            ssems[c % NBUF])

    la_t = (sload(pa_buf, wid * CHUNKS) // 8) * 8
    rdesc = pltpu.async_copy(rw_hbm.at[pl.ds(la_t, TCAP + 16)], r_buf, rsem)

    loads = {}
    stores = {}
    loads[0] = start_load(0)
    if CHUNKS > 1:
        loads[1] = start_load(1)

    rdesc.wait()

    def fbody(k, carry2):
        v = r_buf[pl.ds(k * 16, 16)] >> 1
        fidx_buf[pl.ds(k * 16, 16)] = jnp.maximum(v - (WIN + 1), 0) % N
        return carry2
    lax.fori_loop(0, TCAP // 16, fbody, 0)

    gdescs = [
        pltpu.async_copy(
            x_hbm.at[fidx_buf.at[pl.ds(q * 128, 128)]],
            fill_buf.at[pl.ds(q * 128, 128)],
            gsem,
        )
        for q in range(TCAP // 128)
    ]
    for d in gdescs:
        d.wait()

    def grp_loop(rbuf, fbuf, base, g0, g1, lo_c, hi_c, boff):
        def grp_body(grp, carry3):
            k16 = grp * 16
            rwv = rbuf[pl.ds(k16, 16)]
            rv = rwv >> 1
            rn = rbuf[pl.ds(k16 + 1, 16)] >> 1
            flv = fbuf[pl.ds(k16, 16)]
            ws = jnp.maximum(rv - WIN, lo_c)
            we = jnp.minimum(jnp.minimum(rv, rn - (WIN + 1)), hi_c - 1)
            ln = we - ws + 1
            ok = ((rwv & 1) > 0) & (ln > 0)
            lnz = jnp.where(ok, ln, 0)
            slv = boff + (ws - lo_c)
            for k in range(WIN + 1):
                plsc.store_scatter(data_buf, [slv + k], flv,
                                   mask=lnz > k)
            return carry3
        lax.fori_loop(g0, g1, grp_body, 0)

    def fix_chunk(c):
        g = wid * CHUNKS + c
        lo_c = lo_t + c * S
        hi_c = lo_c + S
        boff = (c % NBUF) * S

        ag = sload(pa_buf, g)
        a2 = sload(pa_buf, g + 2)

        cap = la_t + TCAP
        grp_loop(r_buf, fill_buf, la_t,
                 (ag - la_t) // 16,
                 (jnp.minimum(a2, cap) - la_t + 15) // 16,
                 lo_c, hi_c, boff)

        se0 = cap - 64
        nseg = jnp.where(a2 > se0, (a2 - se0 + PBC - 1) // PBC, 0)

        def seg_body(s, carry):
            so = se0 + s * PBC
            pltpu.sync_copy(rw_hbm.at[pl.ds(so, PBC + 16)], xr_buf)

            def xfbody(k, carry2):
                v = xr_buf[pl.ds(k * 16, 16)] >> 1
                xfidx_buf[pl.ds(k * 16, 16)] = jnp.maximum(
                    v - (WIN + 1), 0) % N
                return carry2
            lax.fori_loop(0, PBC // 16, xfbody, 0)

            descs = [
                pltpu.async_copy(
                    x_hbm.at[xfidx_buf.at[pl.ds(q * 128, 128)]],
                    xfill_buf.at[pl.ds(q * 128, 128)],
                    gsem,
                )
                for q in range(PBC // 128)
            ]
            for d in descs:
                d.wait()

            grp_loop(xr_buf, xfill_buf, so, 0,
                     (jnp.minimum(a2, so + PBC) - so + 15) // 16,
                     lo_c, hi_c, boff)
            return carry

        lax.fori_loop(0, nseg, seg_body, 0)

    for c in range(CHUNKS):
        if c + 2 < CHUNKS:
            if c - 2 >= 0:
                stores[c - 2].wait()
            loads[c + 2] = start_load(c + 2)
        loads[c].wait()
        fix_chunk(c)
        stores[c] = start_store(c)
    for c in range(max(0, CHUNKS - 2), CHUNKS):
        stores[c].wait()


def kernel(x, r_peaks):
    r32 = r_peaks.astype(jnp.int32)
    rw = jnp.concatenate([
        r32 * 2 + _APPLY32,
        SENTINEL + 2 * jnp.arange(PAD_LEN - P, dtype=jnp.int32)])
    los = jnp.arange(NCH + 2, dtype=jnp.int32) * S
    coarse = r32[::256]
    k = jnp.searchsorted(coarse, los, method="compare_all").astype(jnp.int32)
    base = jnp.maximum((k - 1) * 256, 0)
    rpad2 = jnp.pad(r32, (0, 256), constant_values=jnp.int32(2**31 - 1))
    win = rpad2[base[:, None] + jnp.arange(256, dtype=jnp.int32)[None, :]]
    pa = (base + jnp.sum(win < los[:, None], axis=1)).astype(jnp.int32)
    pa = jnp.pad(pa, (0, NCH + 32 - (NCH + 2)), constant_values=P)

    mesh = plsc.VectorSubcoreMesh(core_axis_name="c", subcore_axis_name="s")
    run = functools.partial(
        pl.kernel,
        out_type=jax.ShapeDtypeStruct((N,), jnp.float32),
        mesh=mesh,
        scratch_types=[
            pltpu.VMEM((NBUF * S + 32,), jnp.float32),
            pltpu.VMEM((TCAP + 16,), jnp.int32),
            pltpu.VMEM((TCAP,), jnp.int32),
            pltpu.VMEM((TCAP + 16,), jnp.float32),
            pltpu.VMEM((PBC + 16,), jnp.int32),
            pltpu.VMEM((PBC,), jnp.int32),
            pltpu.VMEM((PBC + 16,), jnp.float32),
            pltpu.VMEM((NCH + 32,), jnp.int32),
        ] + [pltpu.SemaphoreType.DMA] * 10,
        compiler_params=pltpu.CompilerParams(needs_layout_passes=False),
    )(_body)
    return run(x, rw, pa)


# speedup vs baseline: 148.2720x; 1.3131x over previous
import functools

import jax
import jax.numpy as jnp
from jax import lax
from jax.experimental import pallas as pl
from jax.experimental.pallas import tpu as pltpu
from jax.experimental.pallas import tpu_sc as plsc

N = 8_000_000
P = 80_000
RATIO = 0.5
WIN = 20

NC = 2
NS = 16
NW = NC * NS
R = N // NW
CHUNKS = 10
S = R // CHUNKS
NCH = NW * CHUNKS
NBUF = 4

TCAP = 4096
PBC = 768
SENTINEL = 2_000_000_000
PAD_LEN = P + TCAP + 128

_APPLY32 = (jax.random.uniform(jax.random.key(42), (P,)) > RATIO).astype(
    jnp.int32)


def _body(x_hbm, rw_hbm, pa_hbm, out_hbm,
          data_buf, r_buf, fidx_buf, fill_buf, xr_buf, xfidx_buf, xfill_buf,
          pa_buf,
          gsem, rsem, lsem0, lsem1, lsem2, lsem3, ssem0, ssem1, ssem2, ssem3):
    lsems = [lsem0, lsem1, lsem2, lsem3]
    ssems = [ssem0, ssem1, ssem2, ssem3]
    wid = lax.axis_index("s") * NC + lax.axis_index("c")
    lo_t = wid * R

    pltpu.sync_copy(pa_hbm, pa_buf)

    def sload(ref, i):
        return ref[pl.ds(i, 16)][0]

    def start_load(c):
        return pltpu.async_copy(
            x_hbm.at[pl.ds(lo_t + c * S, S)],
            data_buf.at[pl.ds((c % NBUF) * S, S)],
            lsems[c % NBUF])

    def start_store(c):
        return pltpu.async_copy(
            data_buf.at[pl.ds((c % NBUF) * S, S)],
            out_hbm.at[pl.ds(lo_t + c * S, S)],
            ssems[c % NBUF])

    la_t = (sload(pa_buf, wid * CHUNKS) // 8) * 8
    rdesc = pltpu.async_copy(rw_hbm.at[pl.ds(la_t, TCAP + 16)], r_buf, rsem)

    loads = {}
    stores = {}
    loads[0] = start_load(0)
    if CHUNKS > 1:
        loads[1] = start_load(1)

    rdesc.wait()

    def fbody(k, carry2):
        v = r_buf[pl.ds(k * 16, 16)] >> 1
        fidx_buf[pl.ds(k * 16, 16)] = jnp.maximum(v - (WIN + 1), 0) % N
        return carry2
    lax.fori_loop(0, TCAP // 16, fbody, 0)

    gdescs = [
        pltpu.async_copy(
            x_hbm.at[fidx_buf.at[pl.ds(q * 128, 128)]],
            fill_buf.at[pl.ds(q * 128, 128)],
            gsem,
        )
        for q in range(TCAP // 128)
    ]
    for d in gdescs:
        d.wait()

    def grp_loop(rbuf, fbuf, base, g0, g1, lo_c, hi_c, boff):
        def grp_body(grp, carry3):
            k16 = grp * 16
            rwv = rbuf[pl.ds(k16, 16)]
            rv = rwv >> 1
            rn = rbuf[pl.ds(k16 + 1, 16)] >> 1
            flv = fbuf[pl.ds(k16, 16)]
            ws = jnp.maximum(rv - WIN, lo_c)
            we = jnp.minimum(jnp.minimum(rv, rn - (WIN + 1)), hi_c - 1)
            ln = we - ws + 1
            ok = ((rwv & 1) > 0) & (ln > 0)
            lnz = jnp.where(ok, ln, 0)
            slv = boff + (ws - lo_c)
            for k in range(WIN + 1):
                plsc.store_scatter(data_buf, [slv + k], flv,
                                   mask=lnz > k)
            return carry3
        lax.fori_loop(g0, g1, grp_body, 0)

    def fix_chunk(c):
        g = wid * CHUNKS + c
        lo_c = lo_t + c * S
        hi_c = lo_c + S
        boff = (c % NBUF) * S

        ag = sload(pa_buf, g)
        a2 = sload(pa_buf, g + 2) + 256

        cap = la_t + TCAP
        grp_loop(r_buf, fill_buf, la_t,
                 (ag - la_t) // 16,
                 (jnp.minimum(a2, cap) - la_t + 15) // 16,
                 lo_c, hi_c, boff)

        se0 = cap - 64
        nseg = jnp.where(a2 > se0, (a2 - se0 + PBC - 1) // PBC, 0)

        def seg_body(s, carry):
            so = se0 + s * PBC
            pltpu.sync_copy(rw_hbm.at[pl.ds(so, PBC + 16)], xr_buf)

            def xfbody(k, carry2):
                v = xr_buf[pl.ds(k * 16, 16)] >> 1
                xfidx_buf[pl.ds(k * 16, 16)] = jnp.maximum(
                    v - (WIN + 1), 0) % N
                return carry2
            lax.fori_loop(0, PBC // 16, xfbody, 0)

            descs = [
                pltpu.async_copy(
                    x_hbm.at[xfidx_buf.at[pl.ds(q * 128, 128)]],
                    xfill_buf.at[pl.ds(q * 128, 128)],
                    gsem,
                )
                for q in range(PBC // 128)
            ]
            for d in descs:
                d.wait()

            grp_loop(xr_buf, xfill_buf, so, 0,
                     (jnp.minimum(a2, so + PBC) - so + 15) // 16,
                     lo_c, hi_c, boff)
            return carry

        lax.fori_loop(0, nseg, seg_body, 0)

    for c in range(CHUNKS):
        if c + 2 < CHUNKS:
            if c - 2 >= 0:
                stores[c - 2].wait()
            loads[c + 2] = start_load(c + 2)
        loads[c].wait()
        fix_chunk(c)
        stores[c] = start_store(c)
    for c in range(max(0, CHUNKS - 2), CHUNKS):
        stores[c].wait()


def kernel(x, r_peaks):
    r32 = r_peaks.astype(jnp.int32)
    rw = jnp.concatenate([
        r32 * 2 + _APPLY32,
        SENTINEL + 2 * jnp.arange(PAD_LEN - P, dtype=jnp.int32)])
    los = jnp.arange(NCH + 2, dtype=jnp.int32) * S
    coarse = r32[::256]
    k = jnp.searchsorted(coarse, los, method="compare_all").astype(jnp.int32)
    pa = jnp.maximum((k - 1) * 256, 0)
    pa = jnp.pad(pa, (0, NCH + 32 - (NCH + 2)), constant_values=P)

    mesh = plsc.VectorSubcoreMesh(core_axis_name="c", subcore_axis_name="s")
    run = functools.partial(
        pl.kernel,
        out_type=jax.ShapeDtypeStruct((N,), jnp.float32),
        mesh=mesh,
        scratch_types=[
            pltpu.VMEM((NBUF * S + 32,), jnp.float32),
            pltpu.VMEM((TCAP + 16,), jnp.int32),
            pltpu.VMEM((TCAP,), jnp.int32),
            pltpu.VMEM((TCAP + 16,), jnp.float32),
            pltpu.VMEM((PBC + 16,), jnp.int32),
            pltpu.VMEM((PBC,), jnp.int32),
            pltpu.VMEM((PBC + 16,), jnp.float32),
            pltpu.VMEM((NCH + 32,), jnp.int32),
        ] + [pltpu.SemaphoreType.DMA] * 10,
        compiler_params=pltpu.CompilerParams(needs_layout_passes=False),
    )(_body)
    return run(x, rw, pa)


# speedup vs baseline: 148.7227x; 1.0030x over previous
import functools

import jax
import jax.numpy as jnp
from jax import lax
from jax.experimental import pallas as pl
from jax.experimental.pallas import tpu as pltpu
from jax.experimental.pallas import tpu_sc as plsc

N = 8_000_000
P = 80_000
RATIO = 0.5
WIN = 20

NC = 2
NS = 16
NW = NC * NS
R = N // NW
CHUNKS = 10
S = R // CHUNKS
NCH = NW * CHUNKS
NBUF = 4

TCAP = 4096
PBC = 768
SENTINEL = 2_000_000_000
PAD_LEN = P + TCAP + 128

_APPLY32 = (jax.random.uniform(jax.random.key(42), (P,)) > RATIO).astype(
    jnp.int32)


def _body(x_hbm, rw_hbm, pa_hbm, out_hbm,
          data_buf, r_buf, fidx_buf, fill_buf, xr_buf, xfidx_buf, xfill_buf,
          pa_buf,
          gsem, rsem, lsem0, lsem1, lsem2, lsem3, ssem0, ssem1, ssem2, ssem3):
    lsems = [lsem0, lsem1, lsem2, lsem3]
    ssems = [ssem0, ssem1, ssem2, ssem3]
    wid = lax.axis_index("s") * NC + lax.axis_index("c")
    lo_t = wid * R

    pltpu.sync_copy(pa_hbm, pa_buf)

    def sload(ref, i):
        return ref[pl.ds(i, 16)][0]

    def start_load(c):
        return pltpu.async_copy(
            x_hbm.at[pl.ds(lo_t + c * S, S)],
            data_buf.at[pl.ds((c % NBUF) * S, S)],
            lsems[c % NBUF])

    def start_store(c):
        return pltpu.async_copy(
            data_buf.at[pl.ds((c % NBUF) * S, S)],
            out_hbm.at[pl.ds(lo_t + c * S, S)],
            ssems[c % NBUF])

    la_t = (sload(pa_buf, wid * CHUNKS) // 8) * 8
    rdesc = pltpu.async_copy(rw_hbm.at[pl.ds(la_t, TCAP + 16)], r_buf, rsem)

    loads = {}
    stores = {}
    loads[0] = start_load(0)
    if CHUNKS > 1:
        loads[1] = start_load(1)

    rdesc.wait()

    def fbody(k, carry2):
        v = r_buf[pl.ds(k * 16, 16)] >> 1
        fidx_buf[pl.ds(k * 16, 16)] = jnp.maximum(v - (WIN + 1), 0) % N
        return carry2
    lax.fori_loop(0, TCAP // 16, fbody, 0)

    gdescs = [
        pltpu.async_copy(
            x_hbm.at[fidx_buf.at[pl.ds(q * 128, 128)]],
            fill_buf.at[pl.ds(q * 128, 128)],
            gsem,
        )
        for q in range(TCAP // 128)
    ]
    for d in gdescs:
        d.wait()

    def grp_loop(rbuf, fbuf, base, g0, g1, lo_c, hi_c, boff):
        def grp_body(grp, carry3):
            k16 = grp * 16
            rwv = rbuf[pl.ds(k16, 16)]
            rv = rwv >> 1
            rn = rbuf[pl.ds(k16 + 1, 16)] >> 1
            flv = fbuf[pl.ds(k16, 16)]
            ws = jnp.maximum(rv - WIN, lo_c)
            we = jnp.minimum(jnp.minimum(rv, rn - (WIN + 1)), hi_c - 1)
            ln = we - ws + 1
            ok = ((rwv & 1) > 0) & (ln > 0)
            lnz = jnp.where(ok, ln, 0)
            slv = boff + (ws - lo_c)
            for k in range(WIN + 1):
                plsc.store_scatter(data_buf, [slv + k], flv,
                                   mask=lnz > k)
            return carry3
        lax.fori_loop(g0, g1, grp_body, 0)

    def fix_chunk(c):
        g = wid * CHUNKS + c
        lo_c = lo_t + c * S
        hi_c = lo_c + S
        boff = (c % NBUF) * S

        ag = sload(pa_buf, g)
        a2 = sload(pa_buf, g + 2) + 64

        cap = la_t + TCAP
        grp_loop(r_buf, fill_buf, la_t,
                 (ag - la_t) // 16,
                 (jnp.minimum(a2, cap) - la_t + 15) // 16,
                 lo_c, hi_c, boff)

        se0 = cap - 64
        nseg = jnp.where(a2 > se0, (a2 - se0 + PBC - 1) // PBC, 0)

        def seg_body(s, carry):
            so = se0 + s * PBC
            pltpu.sync_copy(rw_hbm.at[pl.ds(so, PBC + 16)], xr_buf)

            def xfbody(k, carry2):
                v = xr_buf[pl.ds(k * 16, 16)] >> 1
                xfidx_buf[pl.ds(k * 16, 16)] = jnp.maximum(
                    v - (WIN + 1), 0) % N
                return carry2
            lax.fori_loop(0, PBC // 16, xfbody, 0)

            descs = [
                pltpu.async_copy(
                    x_hbm.at[xfidx_buf.at[pl.ds(q * 128, 128)]],
                    xfill_buf.at[pl.ds(q * 128, 128)],
                    gsem,
                )
                for q in range(PBC // 128)
            ]
            for d in descs:
                d.wait()

            grp_loop(xr_buf, xfill_buf, so, 0,
                     (jnp.minimum(a2, so + PBC) - so + 15) // 16,
                     lo_c, hi_c, boff)
            return carry

        lax.fori_loop(0, nseg, seg_body, 0)

    for c in range(CHUNKS):
        if c + 2 < CHUNKS:
            if c - 2 >= 0:
                stores[c - 2].wait()
            loads[c + 2] = start_load(c + 2)
        loads[c].wait()
        fix_chunk(c)
        stores[c] = start_store(c)
    for c in range(max(0, CHUNKS - 2), CHUNKS):
        stores[c].wait()


def kernel(x, r_peaks):
    r32 = r_peaks.astype(jnp.int32)
    rw = jnp.concatenate([
        r32 * 2 + _APPLY32,
        SENTINEL + 2 * jnp.arange(PAD_LEN - P, dtype=jnp.int32)])
    los = jnp.arange(NCH + 2, dtype=jnp.int32) * S
    coarse = r32[::64]
    k = jnp.searchsorted(coarse, los, method="compare_all").astype(jnp.int32)
    pa = jnp.maximum((k - 1) * 64, 0)
    pa = jnp.pad(pa, (0, NCH + 32 - (NCH + 2)), constant_values=P)

    mesh = plsc.VectorSubcoreMesh(core_axis_name="c", subcore_axis_name="s")
    run = functools.partial(
        pl.kernel,
        out_type=jax.ShapeDtypeStruct((N,), jnp.float32),
        mesh=mesh,
        scratch_types=[
            pltpu.VMEM((NBUF * S + 32,), jnp.float32),
            pltpu.VMEM((TCAP + 16,), jnp.int32),
            pltpu.VMEM((TCAP,), jnp.int32),
            pltpu.VMEM((TCAP + 16,), jnp.float32),
            pltpu.VMEM((PBC + 16,), jnp.int32),
            pltpu.VMEM((PBC,), jnp.int32),
            pltpu.VMEM((PBC + 16,), jnp.float32),
            pltpu.VMEM((NCH + 32,), jnp.int32),
        ] + [pltpu.SemaphoreType.DMA] * 10,
        compiler_params=pltpu.CompilerParams(needs_layout_passes=False),
    )(_body)
    return run(x, rw, pa)
